# baseline (speedup 1.0000x reference)
import functools

import jax
import jax.numpy as jnp
from jax import lax
from jax.experimental import pallas as pl
from jax.experimental.pallas import tpu as pltpu
from jax.experimental.pallas import tpu_sc as plsc

_N_LOCAL = 10000
_E = 320000
_D = 128
_R = 237

_NC = 2
_NS = 16
_NW = _NC * _NS
_EPW = _E // _NW
_C = 160
_SG = 80
_NCHUNK = -(-_EPW // _C)
_GRP = _C // 16

_mesh = plsc.VectorSubcoreMesh(core_axis_name="c", subcore_axis_name="s")


@functools.partial(
    pl.kernel,
    mesh=_mesh,
    out_type=jax.ShapeDtypeStruct((_E,), jnp.float32),
    compiler_params=pltpu.CompilerParams(needs_layout_passes=False,
                                         use_tc_tiling_on_sc=False),
    scratch_types=[
        pltpu.VMEM((_N_LOCAL,), jnp.int32),
        pltpu.VMEM((_R, _D // 2), jnp.int32),
        pltpu.VMEM((2, _C), jnp.int32),
        pltpu.VMEM((2, _C), jnp.int32),
        pltpu.VMEM((2, _C), jnp.int32),
        pltpu.VMEM((2, _C), jnp.int32),
        pltpu.VMEM((2, _C), jnp.int32),
        pltpu.VMEM((2, _C, _D // 2), jnp.int32),
        pltpu.VMEM((2, _C, _D // 2), jnp.int32),
        pltpu.VMEM((17 * 16,), jnp.float32),
        pltpu.VMEM((2, _C), jnp.float32),
        pltpu.SemaphoreType.DMA((2,)),
        pltpu.SemaphoreType.DMA((2,)),
        pltpu.SemaphoreType.DMA((2,)),
        pltpu.SemaphoreType.DMA((2,)),
        pltpu.SemaphoreType.DMA((2,)),
        pltpu.SemaphoreType.DMA((2,)),
    ],
)
def _sc_score(head_hbm, tail_hbm, rtype_hbm, lgid_hbm, x_hbm, ent_hbm,
              rel_hbm, out_hbm,
              lgid_v, rel_v, hidx_v, tidx_v, ridx_v, gidx_v, rwork_v,
              hrow_v, trow_v, psum_v, out_v,
              semih, semit, semir, semgh, semgt, semo):
    wid = lax.axis_index("s") * _NC + lax.axis_index("c")
    base = wid * _EPW
    pltpu.sync_copy(lgid_hbm, lgid_v)
    pltpu.sync_copy(rel_hbm, rel_v)
    iota = lax.iota(jnp.int32, 16)

    def chunk_off(c):
        return base + jnp.minimum(c * _C, _EPW - _C)

    def issue_idx(c, p, guard=False):
        off = chunk_off(c)

        def go():
            pltpu.async_copy(head_hbm.at[pl.ds(off, _C)], hidx_v.at[p],
                             semih.at[p])
            pltpu.async_copy(tail_hbm.at[pl.ds(off, _C)], tidx_v.at[p],
                             semit.at[p])
            pltpu.async_copy(rtype_hbm.at[pl.ds(off, _C)], ridx_v.at[p],
                             semir.at[p])

        if guard:
            pl.when(c < _NCHUNK)(go)
        else:
            go()

    def stage(p):
        pltpu.make_async_copy(head_hbm.at[pl.ds(0, _C)], hidx_v.at[p],
                              semih.at[p]).wait()
        pltpu.make_async_copy(tail_hbm.at[pl.ds(0, _C)], tidx_v.at[p],
                              semit.at[p]).wait()
        pltpu.make_async_copy(rtype_hbm.at[pl.ds(0, _C)], ridx_v.at[p],
                              semir.at[p]).wait()
        for j in range(_GRP):
            t = tidx_v[p, pl.ds(16 * j, 16)]
            gidx_v[p, pl.ds(16 * j, 16)] = plsc.load_gather(lgid_v, [t])
            rwork_v[p, pl.ds(16 * j, 16)] = ridx_v[p, pl.ds(16 * j, 16)]
        for s in range(_C // _SG):
            sl = pl.ds(s * _SG, _SG)
            pltpu.async_copy(x_hbm.at[hidx_v.at[p, sl]], hrow_v.at[p, sl],
                             semgh.at[p])
            pltpu.async_copy(ent_hbm.at[gidx_v.at[p, sl]], trow_v.at[p, sl],
                             semgt.at[p])

    def wait_gathers(p):
        for s in range(_C // _SG):
            sl = pl.ds(s * _SG, _SG)
            pltpu.make_async_copy(x_hbm.at[hidx_v.at[p, sl]],
                                  hrow_v.at[p, sl], semgh.at[p]).wait()
            pltpu.make_async_copy(ent_hbm.at[gidx_v.at[p, sl]],
                                  trow_v.at[p, sl], semgt.at[p]).wait()

    def wait_out(p):
        pltpu.make_async_copy(out_v.at[p], out_hbm.at[pl.ds(0, _C)],
                              semo.at[p]).wait()

    def compute(c, p):
        def grp_body(gi, carry):
            rvec = rwork_v[p, pl.ds(16 * gi, 16)]
            for l in range(16):
                e = gi * 16 + l
                rid = jnp.take_along_axis(
                    rvec, jnp.full((16,), l, jnp.int32), axis=0)
                acc0 = acc1 = None
                for k in range(4):
                    h = plsc.bitcast(hrow_v[p, e, pl.ds(16 * k, 16)],
                                     jnp.bfloat16)
                    t = plsc.bitcast(trow_v[p, e, pl.ds(16 * k, 16)],
                                     jnp.bfloat16)
                    r = plsc.bitcast(
                        plsc.load_gather(rel_v, [rid, iota + 16 * k]),
                        jnp.bfloat16)
                    u0, u1 = plsc.unpack(
                        jnp.abs(h + r - t),
                        format=plsc.PackFormat.INTERLEAVED)
                    acc0 = u0 if acc0 is None else acc0 + u0
                    acc1 = u1 if acc1 is None else acc1 + u1
                plsc.store_scatter(psum_v, [iota * 17 + l], acc0 + acc1)
            sc = None
            for i in range(16):
                vi = plsc.load_gather(psum_v, [iota + 17 * i])
                sc = vi if sc is None else sc + vi
            out_v[p, pl.ds(gi * 16, 16)] = sc
            return carry

        lax.fori_loop(0, _GRP, grp_body, 0)
        pltpu.async_copy(out_v.at[p], out_hbm.at[pl.ds(chunk_off(c), _C)],
                         semo.at[p])

    def run_iter(c, p, first=False, stage_next=True):
        wait_gathers(p)
        if stage_next:
            issue_idx(c + 2, p, guard=True)
            stage(1 - p)
        if not first:
            wait_out(p)
        compute(c, p)

    issue_idx(0, 0)
    stage(0)
    issue_idx(1, 1)
    run_iter(0, 0, first=True)
    run_iter(1, 1, first=True)

    def pair_body(i, carry):
        c = 2 + 2 * i
        run_iter(c, 0)
        run_iter(c + 1, 1)
        return carry

    lax.fori_loop(0, (_NCHUNK - 3) // 2, pair_body, 0)

    run_iter(_NCHUNK - 1, 0, stage_next=False)
    wait_out(0)
    wait_out(1)


def kernel(x, edge_index, edge_type, local_global_id, ent_emb, rel_emb):
    head = edge_index[0]
    tail = edge_index[1]

    def pack_bf16(a):
        a16 = a.astype(jnp.bfloat16)
        lo = lax.bitcast_convert_type(a16[:, :64], jnp.uint16)
        hi = lax.bitcast_convert_type(a16[:, 64:], jnp.uint16)
        word = lo.astype(jnp.uint32) | (hi.astype(jnp.uint32) << 16)
        return lax.bitcast_convert_type(word, jnp.int32)

    return _sc_score(head, tail, edge_type, local_global_id,
                     pack_bf16(x), pack_bf16(ent_emb), pack_bf16(rel_emb))


# speedup vs baseline: 1.3169x; 1.3169x over previous
import functools

import jax
import jax.numpy as jnp
from jax import lax
from jax.experimental import pallas as pl
from jax.experimental.pallas import tpu as pltpu
from jax.experimental.pallas import tpu_sc as plsc

_N_LOCAL = 10000
_E = 320000
_D = 128
_R = 237

_NC = 2
_NS = 16
_NW = _NC * _NS
_EPW = _E // _NW
_C = 160
_SG = 80
_NCHUNK = -(-_EPW // _C)
_GRP = _C // 16

_mesh = plsc.VectorSubcoreMesh(core_axis_name="c", subcore_axis_name="s")


@functools.partial(
    pl.kernel,
    mesh=_mesh,
    out_type=jax.ShapeDtypeStruct((_E,), jnp.float32),
    compiler_params=pltpu.CompilerParams(needs_layout_passes=False,
                                         use_tc_tiling_on_sc=False),
    scratch_types=[
        pltpu.VMEM((_N_LOCAL,), jnp.int32),
        pltpu.VMEM((_R, _D // 2), jnp.int32),
        pltpu.VMEM((2, _C), jnp.int32),
        pltpu.VMEM((2, _C), jnp.int32),
        pltpu.VMEM((2, _C), jnp.int32),
        pltpu.VMEM((2, _C), jnp.int32),
        pltpu.VMEM((2, _C), jnp.int32),
        pltpu.VMEM((2, _C, _D // 2), jnp.int32),
        pltpu.VMEM((2, _C, _D), jnp.float32),
        pltpu.VMEM((17 * 16,), jnp.float32),
        pltpu.VMEM((2, _C), jnp.float32),
        pltpu.SemaphoreType.DMA((2,)),
        pltpu.SemaphoreType.DMA((2,)),
        pltpu.SemaphoreType.DMA((2,)),
        pltpu.SemaphoreType.DMA((2,)),
        pltpu.SemaphoreType.DMA((2,)),
        pltpu.SemaphoreType.DMA((2,)),
    ],
)
def _sc_score(head_hbm, tail_hbm, rtype_hbm, lgid_hbm, x_hbm, ent_hbm,
              rel_hbm, out_hbm,
              lgid_v, rel_v, hidx_v, tidx_v, ridx_v, gidx_v, rwork_v,
              hrow_v, trow_v, psum_v, out_v,
              semih, semit, semir, semgh, semgt, semo):
    wid = lax.axis_index("s") * _NC + lax.axis_index("c")
    base = wid * _EPW
    pltpu.sync_copy(lgid_hbm, lgid_v)
    pltpu.sync_copy(rel_hbm, rel_v)
    iota = lax.iota(jnp.int32, 16)

    def chunk_off(c):
        return base + jnp.minimum(c * _C, _EPW - _C)

    def issue_idx(c, p, guard=False):
        off = chunk_off(c)

        def go():
            pltpu.async_copy(head_hbm.at[pl.ds(off, _C)], hidx_v.at[p],
                             semih.at[p])
            pltpu.async_copy(tail_hbm.at[pl.ds(off, _C)], tidx_v.at[p],
                             semit.at[p])
            pltpu.async_copy(rtype_hbm.at[pl.ds(off, _C)], ridx_v.at[p],
                             semir.at[p])

        if guard:
            pl.when(c < _NCHUNK)(go)
        else:
            go()

    def stage(p):
        pltpu.make_async_copy(head_hbm.at[pl.ds(0, _C)], hidx_v.at[p],
                              semih.at[p]).wait()
        pltpu.make_async_copy(tail_hbm.at[pl.ds(0, _C)], tidx_v.at[p],
                              semit.at[p]).wait()
        pltpu.make_async_copy(rtype_hbm.at[pl.ds(0, _C)], ridx_v.at[p],
                              semir.at[p]).wait()
        for j in range(_GRP):
            t = tidx_v[p, pl.ds(16 * j, 16)]
            gidx_v[p, pl.ds(16 * j, 16)] = plsc.load_gather(lgid_v, [t])
            rwork_v[p, pl.ds(16 * j, 16)] = ridx_v[p, pl.ds(16 * j, 16)]
        for s in range(_C // _SG):
            sl = pl.ds(s * _SG, _SG)
            pltpu.async_copy(x_hbm.at[hidx_v.at[p, sl]], hrow_v.at[p, sl],
                             semgh.at[p])
            pltpu.async_copy(ent_hbm.at[gidx_v.at[p, sl]], trow_v.at[p, sl],
                             semgt.at[p])

    def wait_gathers(p):
        for s in range(_C // _SG):
            sl = pl.ds(s * _SG, _SG)
            pltpu.make_async_copy(x_hbm.at[hidx_v.at[p, sl]],
                                  hrow_v.at[p, sl], semgh.at[p]).wait()
            pltpu.make_async_copy(ent_hbm.at[gidx_v.at[p, sl]],
                                  trow_v.at[p, sl], semgt.at[p]).wait()

    def wait_out(p):
        pltpu.make_async_copy(out_v.at[p], out_hbm.at[pl.ds(0, _C)],
                              semo.at[p]).wait()

    def compute(c, p):
        def grp_body(gi, carry):
            rvec = rwork_v[p, pl.ds(16 * gi, 16)]
            for l in range(16):
                e = gi * 16 + l
                rid = jnp.take_along_axis(
                    rvec, jnp.full((16,), l, jnp.int32), axis=0)
                acc0 = acc1 = None
                for k in range(4):
                    h = plsc.bitcast(hrow_v[p, e, pl.ds(16 * k, 16)],
                                     jnp.bfloat16)
                    r = plsc.bitcast(
                        plsc.load_gather(rel_v, [rid, iota + 16 * k]),
                        jnp.bfloat16)
                    s0, s1 = plsc.unpack(
                        h + r, format=plsc.PackFormat.INTERLEAVED)
                    t0 = trow_v[p, e, pl.ds(16 * k, 16)]
                    t1 = trow_v[p, e, pl.ds(16 * k + 64, 16)]
                    v0 = jnp.abs(s0 - t0)
                    v1 = jnp.abs(s1 - t1)
                    acc0 = v0 if acc0 is None else acc0 + v0
                    acc1 = v1 if acc1 is None else acc1 + v1
                plsc.store_scatter(psum_v, [iota * 17 + l], acc0 + acc1)
            sc = None
            for i in range(16):
                vi = plsc.load_gather(psum_v, [iota + 17 * i])
                sc = vi if sc is None else sc + vi
            out_v[p, pl.ds(gi * 16, 16)] = sc
            return carry

        lax.fori_loop(0, _GRP, grp_body, 0)
        pltpu.async_copy(out_v.at[p], out_hbm.at[pl.ds(chunk_off(c), _C)],
                         semo.at[p])

    def run_iter(c, p, first=False, stage_next=True):
        wait_gathers(p)
        if stage_next:
            issue_idx(c + 2, p, guard=True)
            stage(1 - p)
        if not first:
            wait_out(p)
        compute(c, p)

    issue_idx(0, 0)
    stage(0)
    issue_idx(1, 1)
    run_iter(0, 0, first=True)
    run_iter(1, 1, first=True)

    def pair_body(i, carry):
        c = 2 + 2 * i
        run_iter(c, 0)
        run_iter(c + 1, 1)
        return carry

    lax.fori_loop(0, (_NCHUNK - 3) // 2, pair_body, 0)

    run_iter(_NCHUNK - 1, 0, stage_next=False)
    wait_out(0)
    wait_out(1)


def kernel(x, edge_index, edge_type, local_global_id, ent_emb, rel_emb):
    head = edge_index[0]
    tail = edge_index[1]

    def pack_bf16(a):
        a16 = a.astype(jnp.bfloat16)
        lo = lax.bitcast_convert_type(a16[:, :64], jnp.uint16)
        hi = lax.bitcast_convert_type(a16[:, 64:], jnp.uint16)
        word = lo.astype(jnp.uint32) | (hi.astype(jnp.uint32) << 16)
        return lax.bitcast_convert_type(word, jnp.int32)

    return _sc_score(head, tail, edge_type, local_global_id,
                     pack_bf16(x), ent_emb, pack_bf16(rel_emb))
